# baseline (speedup 1.0000x reference)
import jax
import jax.numpy as jnp
from jax import lax
from jax.experimental import pallas as pl
from jax.experimental.pallas import tpu as pltpu

_N = 4194304
_L = 128
_RROWS = _N // _L
_IROWS = 2 * _RROWS
_CH = 256
_NB = 16
_NCH = _RROWS // _CH
_NOUT = _NCH // _NB

_COS_C = [0.999999463558197, -19.73903465270996, 64.93061065673828,
          -85.29597473144531, 58.91255569458008, -21.283021926879883]
_SIN_C = [6.283185005187988, -41.341617584228516, 81.60091400146484,
          -76.62655639648438, 41.4034538269043, -12.57640266418457]


def _horner(z, coeffs):
    acc = jnp.float32(coeffs[-1])
    for c in coeffs[-2::-1]:
        acc = acc * z + jnp.float32(c)
    return acc


def _body(ws_ref, rdn_hbm, wo_hbm, g_hbm, out_hbm,
          rdn_v, wo_v, g_v, out_v, rdn_s, wo_s, g_s, out_s):
    w0 = ws_ref[0, 0]
    w1 = ws_ref[0, 1]
    wm = jnp.maximum(w0, w1)
    e0 = jnp.exp(jnp.full((8, _L), w0 - wm, jnp.float32))
    e1 = jnp.exp(jnp.full((8, _L), w1 - wm, jnp.float32))
    p = e1 / (e0 + e1)

    def in_copies(i, b):
        return (
            pltpu.make_async_copy(rdn_hbm.at[pl.ds(i * _CH, _CH)],
                                  rdn_v.at[b], rdn_s.at[b]),
            pltpu.make_async_copy(wo_hbm.at[pl.ds(i * 2 * _CH, 2 * _CH)],
                                  wo_v.at[b], wo_s.at[b]),
            pltpu.make_async_copy(g_hbm.at[pl.ds(i * 2 * _CH, 2 * _CH)],
                                  g_v.at[b], g_s.at[b]),
        )

    def out_copy(i, b):
        return pltpu.make_async_copy(out_v.at[b],
                                     out_hbm.at[pl.ds(i * 2 * _CH, 2 * _CH)],
                                     out_s.at[b])

    for b in range(_NB):
        for c in in_copies(b, b):
            c.start()

    ex = pl.Slice(0, _CH, 2)
    ey = pl.Slice(1, _CH, 2)
    s = jnp.sqrt(jnp.float32(0.1))

    def compute(b):
        rv = rdn_v.at[b]
        wv = wo_v.at[b]
        gv = g_v.at[b]
        ov = out_v.at[b]
        m = rv[...] < p[0:1, :]
        x = wv[ex, :] * 2.0 - 1.0
        y = wv[ey, :] * 2.0 - 1.0
        ax = jnp.abs(x)
        ay = jnp.abs(y)
        cond1 = ax > ay
        nz = jnp.maximum(ax, ay) > 0.0
        r = jnp.where(cond1, x, y)
        num = jnp.where(cond1, y, x)
        den = jnp.where(nz, r, 1.0)
        u2 = (num / den) * 2.0
        w = u2 - jnp.round(u2)
        z = w * w
        cosv = _horner(z, _COS_C)
        sinv = w * _horner(z, _SIN_C)
        ov[ex, :] = jnp.where(m, r * cosv, gv[ex, :] * s)
        ov[ey, :] = jnp.where(m, jnp.where(cond1, r, -r) * sinv,
                              gv[ey, :] * s)

    def outer(g, carry):
        for b in range(_NB):
            i = g * _NB + b
            for c in in_copies(i, b):
                c.wait()

            @pl.when(g > 0)
            def _():
                out_copy(i - _NB, b).wait()

            compute(b)
            out_copy(i, b).start()

            @pl.when(g < _NOUT - 1)
            def _():
                for c in in_copies(i + _NB, b):
                    c.start()
        return carry

    lax.fori_loop(0, _NOUT, outer, 0)
    for b in range(_NB):
        out_copy((_NOUT - 1) * _NB + b, b).wait()


def _pairs_to_rows(a):
    return a.reshape(_RROWS, _L, 2).transpose(0, 2, 1).reshape(_IROWS, _L)


def kernel(weight_scores, rdn, wo, gauss_base):
    rdn2 = rdn.reshape(_RROWS, _L)
    wo2 = _pairs_to_rows(wo)
    g2 = _pairs_to_rows(gauss_base)
    out = pl.pallas_call(
        _body,
        in_specs=[
            pl.BlockSpec(memory_space=pltpu.SMEM),
            pl.BlockSpec(memory_space=pl.ANY),
            pl.BlockSpec(memory_space=pl.ANY),
            pl.BlockSpec(memory_space=pl.ANY),
        ],
        out_specs=pl.BlockSpec(memory_space=pl.ANY),
        out_shape=jax.ShapeDtypeStruct((_IROWS, _L), jnp.float32),
        scratch_shapes=[
            pltpu.VMEM((_NB, _CH, _L), jnp.float32),
            pltpu.VMEM((_NB, 2 * _CH, _L), jnp.float32),
            pltpu.VMEM((_NB, 2 * _CH, _L), jnp.float32),
            pltpu.VMEM((_NB, 2 * _CH, _L), jnp.float32),
            pltpu.SemaphoreType.DMA((_NB,)),
            pltpu.SemaphoreType.DMA((_NB,)),
            pltpu.SemaphoreType.DMA((_NB,)),
            pltpu.SemaphoreType.DMA((_NB,)),
        ],
    )(weight_scores, rdn2, wo2, g2)
    return out.reshape(_RROWS, 2, _L).transpose(0, 2, 1).reshape(_N, 2)


# speedup vs baseline: 1.0040x; 1.0040x over previous
import jax
import jax.numpy as jnp
from jax import lax
from jax.experimental import pallas as pl
from jax.experimental.pallas import tpu as pltpu

_N = 4194304
_L = 128
_RROWS = _N // _L
_IROWS = 2 * _RROWS
_CH = 512
_NB = 8
_NCH = _RROWS // _CH
_NOUT = _NCH // _NB

_COS_C = [0.999999463558197, -19.73903465270996, 64.93061065673828,
          -85.29597473144531, 58.91255569458008, -21.283021926879883]
_SIN_C = [6.283185005187988, -41.341617584228516, 81.60091400146484,
          -76.62655639648438, 41.4034538269043, -12.57640266418457]


def _horner(z, coeffs):
    acc = jnp.float32(coeffs[-1])
    for c in coeffs[-2::-1]:
        acc = acc * z + jnp.float32(c)
    return acc


def _body(ws_ref, rdn_hbm, wo_hbm, g_hbm, out_hbm,
          rdn_v, wo_v, g_v, out_v, rdn_s, wo_s, g_s, out_s):
    w0 = ws_ref[0, 0]
    w1 = ws_ref[0, 1]
    wm = jnp.maximum(w0, w1)
    e0 = jnp.exp(jnp.full((8, _L), w0 - wm, jnp.float32))
    e1 = jnp.exp(jnp.full((8, _L), w1 - wm, jnp.float32))
    p = e1 / (e0 + e1)

    def in_copies(i, b):
        return (
            pltpu.make_async_copy(rdn_hbm.at[pl.ds(i * _CH, _CH)],
                                  rdn_v.at[b], rdn_s.at[b]),
            pltpu.make_async_copy(wo_hbm.at[pl.ds(i * 2 * _CH, 2 * _CH)],
                                  wo_v.at[b], wo_s.at[b]),
            pltpu.make_async_copy(g_hbm.at[pl.ds(i * 2 * _CH, 2 * _CH)],
                                  g_v.at[b], g_s.at[b]),
        )

    def out_copy(i, b):
        return pltpu.make_async_copy(out_v.at[b],
                                     out_hbm.at[pl.ds(i * 2 * _CH, 2 * _CH)],
                                     out_s.at[b])

    for b in range(_NB):
        for c in in_copies(b, b):
            c.start()

    ex = pl.Slice(0, _CH, 2)
    ey = pl.Slice(1, _CH, 2)
    s = jnp.sqrt(jnp.float32(0.1))

    def compute(b):
        rv = rdn_v.at[b]
        wv = wo_v.at[b]
        gv = g_v.at[b]
        ov = out_v.at[b]
        m = rv[...] < p[0:1, :]
        x = wv[ex, :] * 2.0 - 1.0
        y = wv[ey, :] * 2.0 - 1.0
        ax = jnp.abs(x)
        ay = jnp.abs(y)
        cond1 = ax > ay
        nz = jnp.maximum(ax, ay) > 0.0
        r = jnp.where(cond1, x, y)
        num = jnp.where(cond1, y, x)
        den = jnp.where(nz, r, 1.0)
        u2 = (num / den) * 2.0
        w = u2 - jnp.round(u2)
        z = w * w
        cosv = _horner(z, _COS_C)
        sinv = w * _horner(z, _SIN_C)
        ov[ex, :] = jnp.where(m, r * cosv, gv[ex, :] * s)
        ov[ey, :] = jnp.where(m, jnp.where(cond1, r, -r) * sinv,
                              gv[ey, :] * s)

    def outer(g, carry):
        for b in range(_NB):
            i = g * _NB + b
            for c in in_copies(i, b):
                c.wait()

            @pl.when(g > 0)
            def _():
                out_copy(i - _NB, b).wait()

            compute(b)
            out_copy(i, b).start()

            @pl.when(g < _NOUT - 1)
            def _():
                for c in in_copies(i + _NB, b):
                    c.start()
        return carry

    lax.fori_loop(0, _NOUT, outer, 0)
    for b in range(_NB):
        out_copy((_NOUT - 1) * _NB + b, b).wait()


def _pairs_to_rows(a):
    return a.reshape(_RROWS, _L, 2).transpose(0, 2, 1).reshape(_IROWS, _L)


def kernel(weight_scores, rdn, wo, gauss_base):
    rdn2 = rdn.reshape(_RROWS, _L)
    wo2 = _pairs_to_rows(wo)
    g2 = _pairs_to_rows(gauss_base)
    out = pl.pallas_call(
        _body,
        in_specs=[
            pl.BlockSpec(memory_space=pltpu.SMEM),
            pl.BlockSpec(memory_space=pl.ANY),
            pl.BlockSpec(memory_space=pl.ANY),
            pl.BlockSpec(memory_space=pl.ANY),
        ],
        out_specs=pl.BlockSpec(memory_space=pl.ANY),
        out_shape=jax.ShapeDtypeStruct((_IROWS, _L), jnp.float32),
        scratch_shapes=[
            pltpu.VMEM((_NB, _CH, _L), jnp.float32),
            pltpu.VMEM((_NB, 2 * _CH, _L), jnp.float32),
            pltpu.VMEM((_NB, 2 * _CH, _L), jnp.float32),
            pltpu.VMEM((_NB, 2 * _CH, _L), jnp.float32),
            pltpu.SemaphoreType.DMA((_NB,)),
            pltpu.SemaphoreType.DMA((_NB,)),
            pltpu.SemaphoreType.DMA((_NB,)),
            pltpu.SemaphoreType.DMA((_NB,)),
        ],
    )(weight_scores, rdn2, wo2, g2)
    return out.reshape(_RROWS, 2, _L).transpose(0, 2, 1).reshape(_N, 2)
